# Initial kernel scaffold; baseline (speedup 1.0000x reference)
import jax
import jax.numpy as jnp
from jax.experimental import pallas as pl


def kernel(xyz, h, W_a1, b_a1, W_a2, b_a2, W_m1, b_m1, W_m2, b_m2):
    raise NotImplementedError("write your pallas kernel here")


# speedup vs baseline: 2.4907x; 2.4907x over previous
import functools
import jax
import jax.numpy as jnp
from jax import lax
from jax.experimental import pallas as pl

KNB = 16
HIDD = 128
EDGED = HIDD * 2 + 3


def _mlp_kernel(tab_ref, neigh_ref, wac_ref, wan_ref, wax_ref, ba1_ref,
                wa2_ref, wmc_ref, wmn_ref, wmx_ref, bm1_ref, wm2t_ref,
                bm2_ref, out_ref):
    R = tab_ref.shape[0]
    E = R * KNB
    f32 = jnp.float32
    tab = tab_ref[...]
    c_h = tab[:, :HIDD]
    c_xyz = tab[:, HIDD:HIDD + 3]
    ne = neigh_ref[...]
    n_h = ne[:, :HIDD]
    n_xyz = ne[:, HIDD:HIDD + 3]
    dxyz = n_xyz - jnp.broadcast_to(c_xyz[:, None, :], (R, KNB, 3)).reshape(E, 3)

    pa = jnp.dot(c_h, wac_ref[...], preferred_element_type=f32)
    a_pre = (jnp.broadcast_to(pa[:, None, :], (R, KNB, EDGED)).reshape(E, EDGED)
             + jnp.dot(n_h, wan_ref[...], preferred_element_type=f32)
             + ba1_ref[...])
    for d in range(3):
        a_pre = a_pre + dxyz[:, d:d + 1] * wax_ref[d:d + 1, :]
    a = jnp.maximum(a_pre, 0.0)
    scores = jnp.sum(a * wa2_ref[...], axis=1).reshape(R, KNB)
    smax = jnp.max(scores, axis=1, keepdims=True)
    sexp = jnp.exp(scores - smax)
    alpha = sexp / jnp.sum(sexp, axis=1, keepdims=True)

    pm = jnp.dot(c_h, wmc_ref[...], preferred_element_type=f32)
    m_pre = (jnp.broadcast_to(pm[:, None, :], (R, KNB, HIDD)).reshape(E, HIDD)
             + jnp.dot(n_h, wmn_ref[...], preferred_element_type=f32)
             + bm1_ref[...])
    for d in range(3):
        m_pre = m_pre + dxyz[:, d:d + 1] * wmx_ref[d:d + 1, :]
    m = jnp.maximum(m_pre, 0.0)
    msg_flat = jnp.dot(m, wm2t_ref[...], preferred_element_type=f32) + bm2_ref[...]
    wmsg = msg_flat * alpha.reshape(E, 1)
    msg = jnp.sum(wmsg.reshape(R, KNB, HIDD), axis=1)
    out_ref[...] = c_h + msg


def _edge_mlp(table, neigh, W_a1, b_a1, W_a2, b_a2, W_m1, b_m1, W_m2, b_m2,
              interpret=False):
    N = table.shape[0]
    R = 256
    grid = (N // R,)
    wac = (W_a1[:, :HIDD] - W_a1[:, HIDD:2 * HIDD]).T
    wan = W_a1[:, HIDD:2 * HIDD].T
    wax = W_a1[:, 2 * HIDD:].T
    wmc = (W_m1[:, :HIDD] - W_m1[:, HIDD:2 * HIDD]).T
    wmn = W_m1[:, HIDD:2 * HIDD].T
    wmx = W_m1[:, 2 * HIDD:].T
    wm2t = W_m2.T
    full = lambda shape: pl.BlockSpec(shape, lambda i: (0, 0))
    return pl.pallas_call(
        _mlp_kernel,
        grid=grid,
        in_specs=[
            pl.BlockSpec((R, 144), lambda i: (i, 0)),
            pl.BlockSpec((R * KNB, 144), lambda i: (i, 0)),
            full((HIDD, EDGED)),
            full((HIDD, EDGED)),
            full((3, EDGED)),
            full((1, EDGED)),
            full((1, EDGED)),
            full((HIDD, HIDD)),
            full((HIDD, HIDD)),
            full((3, HIDD)),
            full((1, HIDD)),
            full((HIDD, HIDD)),
            full((1, HIDD)),
        ],
        out_specs=pl.BlockSpec((R, HIDD), lambda i: (i, 0)),
        out_shape=jax.ShapeDtypeStruct((N, HIDD), jnp.float32),
        interpret=interpret,
    )(table, neigh, wac, wan, wax, b_a1.reshape(1, EDGED), W_a2,
      wmc, wmn, wmx, b_m1.reshape(1, HIDD), wm2t, b_m2.reshape(1, HIDD))


def kernel(xyz, h, W_a1, b_a1, W_a2, b_a2, W_m1, b_m1, W_m2, b_m2,
           interpret=False):
    B, N, C = h.shape
    xyz2 = xyz[0]
    h2 = h[0]

    sq = jnp.sum(xyz2 * xyz2, axis=-1)
    d2 = sq[:, None] + sq[None, :] - 2.0 * (xyz2 @ xyz2.T)
    idx = lax.top_k(-d2, KNB)[1]

    table = jnp.concatenate(
        [h2, xyz2, jnp.zeros((N, 13), jnp.float32)], axis=1)
    neigh = jnp.take(table, idx.reshape(-1), axis=0)

    out = _edge_mlp(table, neigh, W_a1, b_a1, W_a2, b_a2,
                    W_m1, b_m1, W_m2, b_m2, interpret=interpret)
    return out[None]


# speedup vs baseline: 28.1719x; 11.3108x over previous
import dataclasses
import functools
import jax
import jax.numpy as jnp
from jax import lax
from jax.experimental import pallas as pl
from jax.experimental.pallas import tpu as pltpu
from jax.experimental.pallas import tpu_sc as plsc

KNB = 16
HIDD = 128
EDGED = HIDD * 2 + 3
NCHUNK = 512
LANES = 16
NC, NS = 2, 16
NW = NC * NS


def _chunkmin_kernel(xyzb_ref, xyzt_ref, m_ref, sq_ref):
    R = xyzb_ref.shape[0]
    x0 = xyzb_ref[:, 0:1]
    x1 = xyzb_ref[:, 1:2]
    x2 = xyzb_ref[:, 2:3]
    sqr = (x0 * x0 + x1 * x1) + x2 * x2
    sq_ref[...] = sqr
    y0 = xyzt_ref[0:1, :]
    y1 = xyzt_ref[1:2, :]
    y2 = xyzt_ref[2:3, :]
    sqc = (y0 * y0 + y1 * y1) + y2 * y2
    xb = xyzb_ref[...].astype(jnp.bfloat16)
    yb = xyzt_ref[...].astype(jnp.bfloat16)
    p = jnp.dot(xb, yb, preferred_element_type=jnp.float32)
    d2 = (sqr + sqc) - 2.0 * p
    m_ref[...] = jnp.min(d2.reshape(R, LANES, NCHUNK), axis=1)


def _chunk_mins(xyz2, xyzt, interpret=False):
    N = xyz2.shape[0]
    R = 256
    return pl.pallas_call(
        _chunkmin_kernel,
        grid=(N // R,),
        in_specs=[
            pl.BlockSpec((R, 3), lambda i: (i, 0)),
            pl.BlockSpec((3, N), lambda i: (0, 0)),
        ],
        out_specs=[
            pl.BlockSpec((R, NCHUNK), lambda i: (i, 0)),
            pl.BlockSpec((R, 1), lambda i: (i, 0)),
        ],
        out_shape=[
            jax.ShapeDtypeStruct((N, NCHUNK), jnp.float32),
            jax.ShapeDtypeStruct((N, 1), jnp.float32),
        ],
        interpret=interpret,
    )(xyz2, xyzt)


def _merge16(rk, rv, nk, nv):
    nk2, nv2 = plsc.sort_key_val(nk, nv)
    nk2 = lax.rev(nk2, (0,))
    nv2 = lax.rev(nv2, (0,))
    take_new = nk2 < rk
    mk = jnp.where(take_new, nk2, rk)
    mv = jnp.where(take_new, nv2, rv)
    return plsc.sort_key_val(mk, mv)


def _splat(vec, j):
    dnums = lax.GatherDimensionNumbers(
        offset_dims=(), collapsed_slice_dims=(0,), start_index_map=(0,))
    idxvec = jnp.full((LANES, 1), j, jnp.int32)
    return lax.gather(vec, idxvec, dnums, slice_sizes=(1,),
                      mode=lax.GatherScatterMode.PROMISE_IN_BOUNDS)


def _topk_sc(m, xs, ys, zs, xb, yb, zb, sqt):
    N = m.shape[0]
    rows_per = N // NW
    mesh = plsc.VectorSubcoreMesh(core_axis_name="core",
                                  subcore_axis_name="subcore")

    cp = pltpu.CompilerParams()
    if "needs_layout_passes" in pltpu.CompilerParams.__dataclass_fields__:
        cp = dataclasses.replace(cp, needs_layout_passes=False)

    @functools.partial(
        pl.kernel,
        out_type=(jax.ShapeDtypeStruct((N, KNB), jnp.int32),
                  jax.ShapeDtypeStruct((N, 3 * KNB), jnp.float32)),
        mesh=mesh,
        compiler_params=cp,
        scratch_types=[
            pltpu.VMEM((16, NCHUNK), jnp.float32),
            pltpu.VMEM((N,), jnp.float32),
            pltpu.VMEM((N,), jnp.float32),
            pltpu.VMEM((N,), jnp.float32),
            pltpu.VMEM((N,), jnp.float32),
            pltpu.VMEM((N,), jnp.float32),
            pltpu.VMEM((N,), jnp.float32),
            pltpu.VMEM((N,), jnp.float32),
            pltpu.VMEM((rows_per, KNB), jnp.int32),
            pltpu.VMEM((rows_per, 3 * KNB), jnp.float32),
        ],
    )
    def k(m_hbm, xs_hbm, ys_hbm, zs_hbm, xb_hbm, yb_hbm, zb_hbm, sq_hbm,
          idx_hbm, dxyz_hbm, m_v, xs_v, ys_v, zs_v, xb_v, yb_v, zb_v, sq_v,
          idx_v, dxyz_v):
        wid = lax.axis_index("subcore") * NC + lax.axis_index("core")
        base = wid * rows_per
        pltpu.sync_copy(xs_hbm, xs_v)
        pltpu.sync_copy(ys_hbm, ys_v)
        pltpu.sync_copy(zs_hbm, zs_v)
        pltpu.sync_copy(xb_hbm, xb_v)
        pltpu.sync_copy(yb_hbm, yb_v)
        pltpu.sync_copy(zb_hbm, zb_v)
        pltpu.sync_copy(sq_hbm, sq_v)
        iota = lax.iota(jnp.int32, LANES)
        inf = jnp.full((LANES,), jnp.inf, jnp.float32)
        zero = jnp.zeros((LANES,), jnp.int32)

        @pl.loop(0, rows_per // 16)
        def _(t):
            pltpu.sync_copy(m_hbm.at[pl.ds(base + t * 16, 16)], m_v)

            @pl.loop(0, 16)
            def _(ri):
                r = base + t * 16 + ri
                rk, rv = inf, zero
                for c in range(NCHUNK // LANES):
                    v = m_v[ri, pl.ds(c * LANES, LANES)]
                    rk, rv = _merge16(rk, rv, v, c * LANES + iota)
                rfull = jnp.full((LANES,), r, jnp.int32)
                cxb = plsc.load_gather(xb_v, [rfull])
                cyb = plsc.load_gather(yb_v, [rfull])
                czb = plsc.load_gather(zb_v, [rfull])
                csq = plsc.load_gather(sq_v, [rfull])
                fk, fv = inf, zero
                for j in range(KNB):
                    cj = _splat(rv, j)
                    cols = cj + NCHUNK * iota
                    gxb = plsc.load_gather(xb_v, [cols])
                    gyb = plsc.load_gather(yb_v, [cols])
                    gzb = plsc.load_gather(zb_v, [cols])
                    gsq = plsc.load_gather(sq_v, [cols])
                    p = (gxb * cxb + gyb * cyb) + gzb * czb
                    d2v = (csq + gsq) - 2.0 * p
                    fk, fv = _merge16(fk, fv, d2v, cols)
                idx_v[t * 16 + ri, :] = fv
                cx = plsc.load_gather(xs_v, [rfull])
                cy = plsc.load_gather(ys_v, [rfull])
                cz = plsc.load_gather(zs_v, [rfull])
                gx = plsc.load_gather(xs_v, [fv])
                gy = plsc.load_gather(ys_v, [fv])
                gz = plsc.load_gather(zs_v, [fv])
                dxyz_v[t * 16 + ri, pl.ds(0, LANES)] = gx - cx
                dxyz_v[t * 16 + ri, pl.ds(LANES, LANES)] = gy - cy
                dxyz_v[t * 16 + ri, pl.ds(2 * LANES, LANES)] = gz - cz

        pltpu.sync_copy(idx_v, idx_hbm.at[pl.ds(base, rows_per)])
        pltpu.sync_copy(dxyz_v, dxyz_hbm.at[pl.ds(base, rows_per)])

    return k(m, xs, ys, zs, xb, yb, zb, sqt)


def _gather_sc(table, idx_flat):
    NE = idx_flat.shape[0]
    D = table.shape[1]
    W = 128
    mesh = plsc.VectorSubcoreMesh(core_axis_name="core",
                                  subcore_axis_name="subcore")
    idx2 = idx_flat.reshape(1, NE)

    @functools.partial(
        pl.kernel,
        out_type=jax.ShapeDtypeStruct((NE, D), table.dtype),
        mesh=mesh,
    )
    def k(tab_hbm, i_hbm, o_hbm):
        def body(i_vmem, o_vmem):
            pltpu.sync_copy(tab_hbm.at[i_vmem.at[0]], o_vmem)

        pltpu.emit_pipeline(
            body,
            grid=(NE // W,),
            in_specs=[pl.BlockSpec((1, W), lambda i: (0, i))],
            out_specs=[pl.BlockSpec((W, D), lambda i: (i, 0))],
            core_axis_name=("core", "subcore"),
            dimension_semantics=(pltpu.PARALLEL,),
        )(i_hbm, o_hbm)

    return k(table, idx2)


def _mlp_kernel(h_ref, dxyz_ref, neigh_ref, wac_ref, wan_ref, wax_ref,
                ba1_ref, wa2_ref, wmc_ref, wmn_ref, wmx_ref, bm1_ref,
                wm2t_ref, bm2_ref, out_ref):
    R = h_ref.shape[0]
    E = R * KNB
    f32 = jnp.float32
    c_h = h_ref[...]
    n_h = neigh_ref[...]
    dr = dxyz_ref[...]
    dxyz = [dr[:, d:d + 1] for d in range(3)]

    pa = jnp.dot(c_h, wac_ref[...], preferred_element_type=f32)
    a_pre = (jnp.broadcast_to(pa[:, None, :], (R, KNB, EDGED)).reshape(E, EDGED)
             + jnp.dot(n_h, wan_ref[...], preferred_element_type=f32)
             + ba1_ref[...])
    for d in range(3):
        a_pre = a_pre + dxyz[d] * wax_ref[d:d + 1, :]
    a = jnp.maximum(a_pre, 0.0)
    scores = jnp.sum(a * wa2_ref[...], axis=1).reshape(R, KNB)
    smax = jnp.max(scores, axis=1, keepdims=True)
    sexp = jnp.exp(scores - smax)
    alpha = sexp / jnp.sum(sexp, axis=1, keepdims=True)

    pm = jnp.dot(c_h, wmc_ref[...], preferred_element_type=f32)
    m_pre = (jnp.broadcast_to(pm[:, None, :], (R, KNB, HIDD)).reshape(E, HIDD)
             + jnp.dot(n_h, wmn_ref[...], preferred_element_type=f32)
             + bm1_ref[...])
    for d in range(3):
        m_pre = m_pre + dxyz[d] * wmx_ref[d:d + 1, :]
    m = jnp.maximum(m_pre, 0.0)
    msg_flat = jnp.dot(m, wm2t_ref[...], preferred_element_type=f32) + bm2_ref[...]
    wmsg = msg_flat * alpha.reshape(E, 1)
    msg = jnp.sum(wmsg.reshape(R, KNB, HIDD), axis=1)
    out_ref[...] = c_h + msg


def _edge_mlp(h2, dxyz, neigh, W_a1, b_a1, W_a2, b_a2, W_m1, b_m1, W_m2,
              b_m2, interpret=False):
    N = h2.shape[0]
    R = 256
    grid = (N // R,)
    wac = (W_a1[:, :HIDD] - W_a1[:, HIDD:2 * HIDD]).T
    wan = W_a1[:, HIDD:2 * HIDD].T
    wax = W_a1[:, 2 * HIDD:].T
    wmc = (W_m1[:, :HIDD] - W_m1[:, HIDD:2 * HIDD]).T
    wmn = W_m1[:, HIDD:2 * HIDD].T
    wmx = W_m1[:, 2 * HIDD:].T
    wm2t = W_m2.T
    full = lambda shape: pl.BlockSpec(shape, lambda i: (0, 0))
    return pl.pallas_call(
        _mlp_kernel,
        grid=grid,
        in_specs=[
            pl.BlockSpec((R, HIDD), lambda i: (i, 0)),
            pl.BlockSpec((R * KNB, 3), lambda i: (i, 0)),
            pl.BlockSpec((R * KNB, HIDD), lambda i: (i, 0)),
            full((HIDD, EDGED)),
            full((HIDD, EDGED)),
            full((3, EDGED)),
            full((1, EDGED)),
            full((1, EDGED)),
            full((HIDD, HIDD)),
            full((HIDD, HIDD)),
            full((3, HIDD)),
            full((1, HIDD)),
            full((HIDD, HIDD)),
            full((1, HIDD)),
        ],
        out_specs=pl.BlockSpec((R, HIDD), lambda i: (i, 0)),
        out_shape=jax.ShapeDtypeStruct((N, HIDD), jnp.float32),
        interpret=interpret,
    )(h2, dxyz, neigh, wac, wan, wax, b_a1.reshape(1, EDGED), W_a2,
      wmc, wmn, wmx, b_m1.reshape(1, HIDD), wm2t, b_m2.reshape(1, HIDD))


def kernel(xyz, h, W_a1, b_a1, W_a2, b_a2, W_m1, b_m1, W_m2, b_m2):
    B, N, C = h.shape
    xyz2 = xyz[0]
    h2 = h[0]
    xyzt = xyz2.T

    m, sqt = _chunk_mins(xyz2, xyzt)
    xyzt_b = xyzt.astype(jnp.bfloat16).astype(jnp.float32)
    idx, dxyz = _topk_sc(m, xyzt[0], xyzt[1], xyzt[2],
                         xyzt_b[0], xyzt_b[1], xyzt_b[2],
                         sqt.reshape(N))
    neigh = _gather_sc(h2, idx.reshape(-1))
    dxyz_e = dxyz.reshape(N, 3, KNB).transpose(0, 2, 1).reshape(N * KNB, 3)

    out = _edge_mlp(h2, dxyz_e, neigh, W_a1, b_a1, W_a2, b_a2,
                    W_m1, b_m1, W_m2, b_m2)
    return out[None]


# speedup vs baseline: 29.0912x; 1.0326x over previous
import dataclasses
import functools
import jax
import jax.numpy as jnp
from jax import lax
from jax.experimental import pallas as pl
from jax.experimental.pallas import tpu as pltpu
from jax.experimental.pallas import tpu_sc as plsc

KNB = 16
HIDD = 128
EDGED = HIDD * 2 + 3
NCHUNK = 512
LANES = 16
NC, NS = 2, 16
NW = NC * NS


def _chunkmin_kernel(xyzb_ref, xyzt_ref, m_ref, d2_ref):
    R = xyzb_ref.shape[0]
    NN = xyzt_ref.shape[1]
    x0 = xyzb_ref[:, 0:1]
    x1 = xyzb_ref[:, 1:2]
    x2 = xyzb_ref[:, 2:3]
    sqr = (x0 * x0 + x1 * x1) + x2 * x2
    y0 = xyzt_ref[0:1, :]
    y1 = xyzt_ref[1:2, :]
    y2 = xyzt_ref[2:3, :]
    sqc = (y0 * y0 + y1 * y1) + y2 * y2
    xb = xyzb_ref[...].astype(jnp.bfloat16)
    yb = xyzt_ref[...].astype(jnp.bfloat16)
    p = jnp.dot(xb, yb, preferred_element_type=jnp.float32)
    d2 = (sqr + sqc) - 2.0 * p
    d2_ref[...] = d2
    m_ref[...] = jnp.min(d2.reshape(R, LANES, NCHUNK), axis=1)


def _chunk_mins(xyz2, xyzt, interpret=False):
    N = xyz2.shape[0]
    R = 256
    return pl.pallas_call(
        _chunkmin_kernel,
        grid=(N // R,),
        in_specs=[
            pl.BlockSpec((R, 3), lambda i: (i, 0)),
            pl.BlockSpec((3, N), lambda i: (0, 0)),
        ],
        out_specs=[
            pl.BlockSpec((R, NCHUNK), lambda i: (i, 0)),
            pl.BlockSpec((R, N), lambda i: (i, 0)),
        ],
        out_shape=[
            jax.ShapeDtypeStruct((N, NCHUNK), jnp.float32),
            jax.ShapeDtypeStruct((N, N), jnp.float32),
        ],
        interpret=interpret,
    )(xyz2, xyzt)


def _merge16(rk, rv, nk, nv):
    nk2, nv2 = plsc.sort_key_val(nk, nv)
    nk2 = lax.rev(nk2, (0,))
    nv2 = lax.rev(nv2, (0,))
    take_new = nk2 < rk
    mk = jnp.where(take_new, nk2, rk)
    mv = jnp.where(take_new, nv2, rv)
    return plsc.sort_key_val(mk, mv)


def _bf16_rtne(x):
    u = lax.bitcast_convert_type(x, jnp.uint32)
    u = u + jnp.uint32(0x7FFF) + ((u >> 16) & jnp.uint32(1))
    return lax.bitcast_convert_type(u & jnp.uint32(0xFFFF0000), jnp.float32)


def _splat(vec, j):
    dnums = lax.GatherDimensionNumbers(
        offset_dims=(), collapsed_slice_dims=(0,), start_index_map=(0,))
    idxvec = jnp.full((LANES, 1), j, jnp.int32)
    return lax.gather(vec, idxvec, dnums, slice_sizes=(1,),
                      mode=lax.GatherScatterMode.PROMISE_IN_BOUNDS)


def _topk_sc(m, d2, xs, ys, zs):
    N = m.shape[0]
    rows_per = N // NW
    mesh = plsc.VectorSubcoreMesh(core_axis_name="core",
                                  subcore_axis_name="subcore")

    cp = pltpu.CompilerParams()
    if "needs_layout_passes" in pltpu.CompilerParams.__dataclass_fields__:
        cp = dataclasses.replace(cp, needs_layout_passes=False)

    @functools.partial(
        pl.kernel,
        out_type=(jax.ShapeDtypeStruct((N, KNB), jnp.int32),
                  jax.ShapeDtypeStruct((N, 3 * KNB), jnp.float32)),
        mesh=mesh,
        compiler_params=cp,
        scratch_types=[
            pltpu.VMEM((16, NCHUNK), jnp.float32),
            pltpu.VMEM((N,), jnp.float32),
            pltpu.VMEM((N,), jnp.float32),
            pltpu.VMEM((N,), jnp.float32),
            pltpu.VMEM((N,), jnp.float32),
            pltpu.VMEM((N,), jnp.float32),
            pltpu.VMEM((rows_per, KNB), jnp.int32),
            pltpu.VMEM((rows_per, 3 * KNB), jnp.float32),
            pltpu.SemaphoreType.DMA,
            pltpu.SemaphoreType.DMA,
        ],
    )
    def k(m_hbm, d2_hbm, xs_hbm, ys_hbm, zs_hbm, idx_hbm, dxyz_hbm,
          m_v, row_a, row_b, xs_v, ys_v, zs_v, idx_v, dxyz_v, sem_a, sem_b):
        wid = lax.axis_index("subcore") * NC + lax.axis_index("core")
        base = wid * rows_per
        pltpu.sync_copy(xs_hbm, xs_v)
        pltpu.sync_copy(ys_hbm, ys_v)
        pltpu.sync_copy(zs_hbm, zs_v)
        iota = lax.iota(jnp.int32, LANES)
        inf = jnp.full((LANES,), jnp.inf, jnp.float32)
        zero = jnp.zeros((LANES,), jnp.int32)

        def process(gl, buf):
            rk, rv = inf, zero
            for c in range(NCHUNK // LANES):
                v = m_v[gl % 16, pl.ds(c * LANES, LANES)] + 4.0
                rk, rv = _merge16(rk, rv, v, c * LANES + iota)
            fk, fv = inf, zero
            for j in range(KNB):
                cj = _splat(rv, j)
                cols = cj + NCHUNK * iota
                d2v = plsc.load_gather(buf, [cols])
                fk, fv = _merge16(fk, fv, d2v + 4.0, cols)
            idx_v[gl, :] = fv
            r = base + gl
            rfull = jnp.full((LANES,), r, jnp.int32)
            cx = plsc.load_gather(xs_v, [rfull])
            cy = plsc.load_gather(ys_v, [rfull])
            cz = plsc.load_gather(zs_v, [rfull])
            gx = plsc.load_gather(xs_v, [fv])
            gy = plsc.load_gather(ys_v, [fv])
            gz = plsc.load_gather(zs_v, [fv])
            dxyz_v[gl, pl.ds(0, LANES)] = gx - cx
            dxyz_v[gl, pl.ds(LANES, LANES)] = gy - cy
            dxyz_v[gl, pl.ds(2 * LANES, LANES)] = gz - cz

        @pl.loop(0, rows_per // 16)
        def _(t):
            pltpu.sync_copy(m_hbm.at[pl.ds(base + t * 16, 16)], m_v)
            pltpu.async_copy(d2_hbm.at[base + t * 16], row_a, sem_a)

            @pl.loop(0, 8)
            def _(p):
                g0 = t * 16 + 2 * p
                r0 = base + g0
                pltpu.make_async_copy(d2_hbm.at[r0], row_a, sem_a).wait()
                pltpu.async_copy(d2_hbm.at[r0 + 1], row_b, sem_b)
                process(g0, row_a)
                pltpu.make_async_copy(d2_hbm.at[r0 + 1], row_b, sem_b).wait()

                @pl.when(2 * p + 2 < 16)
                def _():
                    pltpu.async_copy(d2_hbm.at[r0 + 2], row_a, sem_a)

                process(g0 + 1, row_b)

        pltpu.sync_copy(idx_v, idx_hbm.at[pl.ds(base, rows_per)])
        pltpu.sync_copy(dxyz_v, dxyz_hbm.at[pl.ds(base, rows_per)])

    return k(m, d2, xs, ys, zs)


def _gather_sc(table, idx_flat):
    NE = idx_flat.shape[0]
    D = table.shape[1]
    W = 128
    mesh = plsc.VectorSubcoreMesh(core_axis_name="core",
                                  subcore_axis_name="subcore")
    idx2 = idx_flat.reshape(1, NE)

    @functools.partial(
        pl.kernel,
        out_type=jax.ShapeDtypeStruct((NE, D), table.dtype),
        mesh=mesh,
    )
    def k(tab_hbm, i_hbm, o_hbm):
        def body(i_vmem, o_vmem):
            pltpu.sync_copy(tab_hbm.at[i_vmem.at[0]], o_vmem)

        pltpu.emit_pipeline(
            body,
            grid=(NE // W,),
            in_specs=[pl.BlockSpec((1, W), lambda i: (0, i))],
            out_specs=[pl.BlockSpec((W, D), lambda i: (i, 0))],
            core_axis_name=("core", "subcore"),
            dimension_semantics=(pltpu.PARALLEL,),
        )(i_hbm, o_hbm)

    return k(table, idx2)


def _mlp_kernel(h_ref, dxyz_ref, neigh_ref, wac_ref, wan_ref, wax_ref,
                ba1_ref, wa2_ref, wmc_ref, wmn_ref, wmx_ref, bm1_ref,
                wm2t_ref, bm2_ref, out_ref):
    R = h_ref.shape[0]
    E = R * KNB
    f32 = jnp.float32
    c_h = h_ref[...]
    n_h = neigh_ref[...]
    dr = dxyz_ref[...]
    dxyz = [dr[:, d:d + 1] for d in range(3)]

    pa = jnp.dot(c_h, wac_ref[...], preferred_element_type=f32)
    a_pre = (jnp.broadcast_to(pa[:, None, :], (R, KNB, EDGED)).reshape(E, EDGED)
             + jnp.dot(n_h, wan_ref[...], preferred_element_type=f32)
             + ba1_ref[...])
    for d in range(3):
        a_pre = a_pre + dxyz[d] * wax_ref[d:d + 1, :]
    a = jnp.maximum(a_pre, 0.0)
    scores = jnp.sum(a * wa2_ref[...], axis=1).reshape(R, KNB)
    smax = jnp.max(scores, axis=1, keepdims=True)
    sexp = jnp.exp(scores - smax)
    alpha = sexp / jnp.sum(sexp, axis=1, keepdims=True)

    pm = jnp.dot(c_h, wmc_ref[...], preferred_element_type=f32)
    m_pre = (jnp.broadcast_to(pm[:, None, :], (R, KNB, HIDD)).reshape(E, HIDD)
             + jnp.dot(n_h, wmn_ref[...], preferred_element_type=f32)
             + bm1_ref[...])
    for d in range(3):
        m_pre = m_pre + dxyz[d] * wmx_ref[d:d + 1, :]
    m = jnp.maximum(m_pre, 0.0)
    msg_flat = jnp.dot(m, wm2t_ref[...], preferred_element_type=f32) + bm2_ref[...]
    wmsg = msg_flat * alpha.reshape(E, 1)
    msg = jnp.sum(wmsg.reshape(R, KNB, HIDD), axis=1)
    out_ref[...] = c_h + msg


def _edge_mlp(h2, dxyz, neigh, W_a1, b_a1, W_a2, b_a2, W_m1, b_m1, W_m2,
              b_m2, interpret=False):
    N = h2.shape[0]
    R = 256
    grid = (N // R,)
    wac = (W_a1[:, :HIDD] - W_a1[:, HIDD:2 * HIDD]).T
    wan = W_a1[:, HIDD:2 * HIDD].T
    wax = W_a1[:, 2 * HIDD:].T
    wmc = (W_m1[:, :HIDD] - W_m1[:, HIDD:2 * HIDD]).T
    wmn = W_m1[:, HIDD:2 * HIDD].T
    wmx = W_m1[:, 2 * HIDD:].T
    wm2t = W_m2.T
    full = lambda shape: pl.BlockSpec(shape, lambda i: (0, 0))
    return pl.pallas_call(
        _mlp_kernel,
        grid=grid,
        in_specs=[
            pl.BlockSpec((R, HIDD), lambda i: (i, 0)),
            pl.BlockSpec((R * KNB, 3), lambda i: (i, 0)),
            pl.BlockSpec((R * KNB, HIDD), lambda i: (i, 0)),
            full((HIDD, EDGED)),
            full((HIDD, EDGED)),
            full((3, EDGED)),
            full((1, EDGED)),
            full((1, EDGED)),
            full((HIDD, HIDD)),
            full((HIDD, HIDD)),
            full((3, HIDD)),
            full((1, HIDD)),
            full((HIDD, HIDD)),
            full((1, HIDD)),
        ],
        out_specs=pl.BlockSpec((R, HIDD), lambda i: (i, 0)),
        out_shape=jax.ShapeDtypeStruct((N, HIDD), jnp.float32),
        interpret=interpret,
    )(h2, dxyz, neigh, wac, wan, wax, b_a1.reshape(1, EDGED), W_a2,
      wmc, wmn, wmx, b_m1.reshape(1, HIDD), wm2t, b_m2.reshape(1, HIDD))


def kernel(xyz, h, W_a1, b_a1, W_a2, b_a2, W_m1, b_m1, W_m2, b_m2):
    B, N, C = h.shape
    xyz2 = xyz[0]
    h2 = h[0]
    xyzt = xyz2.T

    m, d2 = _chunk_mins(xyz2, xyzt)
    idx, dxyz = _topk_sc(m, d2, xyzt[0], xyzt[1], xyzt[2])
    neigh = _gather_sc(h2, idx.reshape(-1))
    dxyz_e = dxyz.reshape(N, 3, KNB).transpose(0, 2, 1).reshape(N * KNB, 3)

    out = _edge_mlp(h2, dxyz_e, neigh, W_a1, b_a1, W_a2, b_a2,
                    W_m1, b_m1, W_m2, b_m2)
    return out[None]


# speedup vs baseline: 29.3685x; 1.0095x over previous
import dataclasses
import functools
import jax
import jax.numpy as jnp
from jax import lax
from jax.experimental import pallas as pl
from jax.experimental.pallas import tpu as pltpu
from jax.experimental.pallas import tpu_sc as plsc

KNB = 16
HIDD = 128
EDGED = HIDD * 2 + 3
NCHUNK = 512
LANES = 16
NC, NS = 2, 16
NW = NC * NS


def _chunkmin_kernel(xyzb_ref, xyzt_ref, m_ref, d2_ref):
    R = xyzb_ref.shape[0]
    NN = xyzt_ref.shape[1]
    x0 = xyzb_ref[:, 0:1]
    x1 = xyzb_ref[:, 1:2]
    x2 = xyzb_ref[:, 2:3]
    sqr = (x0 * x0 + x1 * x1) + x2 * x2
    y0 = xyzt_ref[0:1, :]
    y1 = xyzt_ref[1:2, :]
    y2 = xyzt_ref[2:3, :]
    sqc = (y0 * y0 + y1 * y1) + y2 * y2
    xb = xyzb_ref[...].astype(jnp.bfloat16)
    yb = xyzt_ref[...].astype(jnp.bfloat16)
    p = jnp.dot(xb, yb, preferred_element_type=jnp.float32)
    d2 = (sqr + sqc) - 2.0 * p
    d2_ref[...] = d2
    m_ref[...] = jnp.min(d2.reshape(R, LANES, NCHUNK), axis=1)


def _chunk_mins(xyz2, xyzt, interpret=False):
    N = xyz2.shape[0]
    R = 256
    return pl.pallas_call(
        _chunkmin_kernel,
        grid=(N // R,),
        in_specs=[
            pl.BlockSpec((R, 3), lambda i: (i, 0)),
            pl.BlockSpec((3, N), lambda i: (0, 0)),
        ],
        out_specs=[
            pl.BlockSpec((R, NCHUNK), lambda i: (i, 0)),
            pl.BlockSpec((R, N), lambda i: (i, 0)),
        ],
        out_shape=[
            jax.ShapeDtypeStruct((N, NCHUNK), jnp.float32),
            jax.ShapeDtypeStruct((N, N), jnp.float32),
        ],
        interpret=interpret,
    )(xyz2, xyzt)


def _merge_sorted(ak, av, bk, bv):
    bk2 = lax.rev(bk, (0,))
    bv2 = lax.rev(bv, (0,))
    take_b = bk2 < ak
    mk = jnp.where(take_b, bk2, ak)
    mv = jnp.where(take_b, bv2, av)
    return plsc.sort_key_val(mk, mv)


def _tree_top16(leaves):
    while len(leaves) > 1:
        nxt = []
        for i in range(0, len(leaves) - 1, 2):
            nxt.append(_merge_sorted(*leaves[i], *leaves[i + 1]))
        if len(leaves) % 2:
            nxt.append(leaves[-1])
        leaves = nxt
    return leaves[0]


def _bf16_rtne(x):
    u = lax.bitcast_convert_type(x, jnp.uint32)
    u = u + jnp.uint32(0x7FFF) + ((u >> 16) & jnp.uint32(1))
    return lax.bitcast_convert_type(u & jnp.uint32(0xFFFF0000), jnp.float32)


def _splat(vec, j):
    dnums = lax.GatherDimensionNumbers(
        offset_dims=(), collapsed_slice_dims=(0,), start_index_map=(0,))
    idxvec = jnp.full((LANES, 1), j, jnp.int32)
    return lax.gather(vec, idxvec, dnums, slice_sizes=(1,),
                      mode=lax.GatherScatterMode.PROMISE_IN_BOUNDS)


def _topk_sc(m, d2, xs, ys, zs):
    N = m.shape[0]
    rows_per = N // NW
    mesh = plsc.VectorSubcoreMesh(core_axis_name="core",
                                  subcore_axis_name="subcore")

    cp = pltpu.CompilerParams()
    if "needs_layout_passes" in pltpu.CompilerParams.__dataclass_fields__:
        cp = dataclasses.replace(cp, needs_layout_passes=False)

    @functools.partial(
        pl.kernel,
        out_type=(jax.ShapeDtypeStruct((N, KNB), jnp.int32),
                  jax.ShapeDtypeStruct((N, 3 * KNB), jnp.float32)),
        mesh=mesh,
        compiler_params=cp,
        scratch_types=[
            pltpu.VMEM((16, NCHUNK), jnp.float32),
            pltpu.VMEM((N,), jnp.float32),
            pltpu.VMEM((N,), jnp.float32),
            pltpu.VMEM((N,), jnp.float32),
            pltpu.VMEM((N,), jnp.float32),
            pltpu.VMEM((N,), jnp.float32),
            pltpu.VMEM((rows_per, KNB), jnp.int32),
            pltpu.VMEM((rows_per, 3 * KNB), jnp.float32),
            pltpu.SemaphoreType.DMA,
            pltpu.SemaphoreType.DMA,
        ],
    )
    def k(m_hbm, d2_hbm, xs_hbm, ys_hbm, zs_hbm, idx_hbm, dxyz_hbm,
          m_v, row_a, row_b, xs_v, ys_v, zs_v, idx_v, dxyz_v, sem_a, sem_b):
        wid = lax.axis_index("subcore") * NC + lax.axis_index("core")
        base = wid * rows_per
        pltpu.sync_copy(xs_hbm, xs_v)
        pltpu.sync_copy(ys_hbm, ys_v)
        pltpu.sync_copy(zs_hbm, zs_v)
        iota = lax.iota(jnp.int32, LANES)
        inf = jnp.full((LANES,), jnp.inf, jnp.float32)
        zero = jnp.zeros((LANES,), jnp.int32)

        def process(gl, buf):
            leaves = []
            for c in range(NCHUNK // LANES):
                v = m_v[gl % 16, pl.ds(c * LANES, LANES)]
                leaves.append(plsc.sort_key_val(v, c * LANES + iota))
            rk, rv = _tree_top16(leaves)
            leaves = []
            for j in range(KNB):
                cj = _splat(rv, j)
                cols = cj + NCHUNK * iota
                d2v = plsc.load_gather(buf, [cols])
                leaves.append(plsc.sort_key_val(d2v, cols))
            fk, fv = _tree_top16(leaves)
            idx_v[gl, :] = fv
            r = base + gl
            rfull = jnp.full((LANES,), r, jnp.int32)
            cx = plsc.load_gather(xs_v, [rfull])
            cy = plsc.load_gather(ys_v, [rfull])
            cz = plsc.load_gather(zs_v, [rfull])
            gx = plsc.load_gather(xs_v, [fv])
            gy = plsc.load_gather(ys_v, [fv])
            gz = plsc.load_gather(zs_v, [fv])
            dxyz_v[gl, pl.ds(0, LANES)] = gx - cx
            dxyz_v[gl, pl.ds(LANES, LANES)] = gy - cy
            dxyz_v[gl, pl.ds(2 * LANES, LANES)] = gz - cz

        @pl.loop(0, rows_per // 16)
        def _(t):
            pltpu.sync_copy(m_hbm.at[pl.ds(base + t * 16, 16)], m_v)
            pltpu.async_copy(d2_hbm.at[base + t * 16], row_a, sem_a)

            @pl.loop(0, 8)
            def _(p):
                g0 = t * 16 + 2 * p
                r0 = base + g0
                pltpu.make_async_copy(d2_hbm.at[r0], row_a, sem_a).wait()
                pltpu.async_copy(d2_hbm.at[r0 + 1], row_b, sem_b)
                process(g0, row_a)
                pltpu.make_async_copy(d2_hbm.at[r0 + 1], row_b, sem_b).wait()

                @pl.when(2 * p + 2 < 16)
                def _():
                    pltpu.async_copy(d2_hbm.at[r0 + 2], row_a, sem_a)

                process(g0 + 1, row_b)

        pltpu.sync_copy(idx_v, idx_hbm.at[pl.ds(base, rows_per)])
        pltpu.sync_copy(dxyz_v, dxyz_hbm.at[pl.ds(base, rows_per)])

    return k(m, d2, xs, ys, zs)


def _gather_sc(table, idx_flat):
    NE = idx_flat.shape[0]
    D = table.shape[1]
    W = 128
    mesh = plsc.VectorSubcoreMesh(core_axis_name="core",
                                  subcore_axis_name="subcore")
    idx2 = idx_flat.reshape(1, NE)

    @functools.partial(
        pl.kernel,
        out_type=jax.ShapeDtypeStruct((NE, D), table.dtype),
        mesh=mesh,
    )
    def k(tab_hbm, i_hbm, o_hbm):
        def body(i_vmem, o_vmem):
            pltpu.sync_copy(tab_hbm.at[i_vmem.at[0]], o_vmem)

        pltpu.emit_pipeline(
            body,
            grid=(NE // W,),
            in_specs=[pl.BlockSpec((1, W), lambda i: (0, i))],
            out_specs=[pl.BlockSpec((W, D), lambda i: (i, 0))],
            core_axis_name=("core", "subcore"),
            dimension_semantics=(pltpu.PARALLEL,),
        )(i_hbm, o_hbm)

    return k(table, idx2)


def _mlp_kernel(h_ref, dxyz_ref, neigh_ref, wac_ref, wan_ref, wax_ref,
                ba1_ref, wa2_ref, wmc_ref, wmn_ref, wmx_ref, bm1_ref,
                wm2t_ref, bm2_ref, out_ref):
    R = h_ref.shape[0]
    E = R * KNB
    f32 = jnp.float32
    c_h = h_ref[...]
    n_h = neigh_ref[...]
    dr = dxyz_ref[...]
    dxyz = [dr[:, d:d + 1] for d in range(3)]

    pa = jnp.dot(c_h, wac_ref[...], preferred_element_type=f32)
    a_pre = (jnp.broadcast_to(pa[:, None, :], (R, KNB, EDGED)).reshape(E, EDGED)
             + jnp.dot(n_h, wan_ref[...], preferred_element_type=f32)
             + ba1_ref[...])
    for d in range(3):
        a_pre = a_pre + dxyz[d] * wax_ref[d:d + 1, :]
    a = jnp.maximum(a_pre, 0.0)
    scores = jnp.sum(a * wa2_ref[...], axis=1).reshape(R, KNB)
    smax = jnp.max(scores, axis=1, keepdims=True)
    sexp = jnp.exp(scores - smax)
    alpha = sexp / jnp.sum(sexp, axis=1, keepdims=True)

    pm = jnp.dot(c_h, wmc_ref[...], preferred_element_type=f32)
    m_pre = (jnp.broadcast_to(pm[:, None, :], (R, KNB, HIDD)).reshape(E, HIDD)
             + jnp.dot(n_h, wmn_ref[...], preferred_element_type=f32)
             + bm1_ref[...])
    for d in range(3):
        m_pre = m_pre + dxyz[d] * wmx_ref[d:d + 1, :]
    m = jnp.maximum(m_pre, 0.0)
    msg_flat = jnp.dot(m, wm2t_ref[...], preferred_element_type=f32) + bm2_ref[...]
    wmsg = msg_flat * alpha.reshape(E, 1)
    msg = jnp.sum(wmsg.reshape(R, KNB, HIDD), axis=1)
    out_ref[...] = c_h + msg


def _edge_mlp(h2, dxyz, neigh, W_a1, b_a1, W_a2, b_a2, W_m1, b_m1, W_m2,
              b_m2, interpret=False):
    N = h2.shape[0]
    R = 256
    grid = (N // R,)
    wac = (W_a1[:, :HIDD] - W_a1[:, HIDD:2 * HIDD]).T
    wan = W_a1[:, HIDD:2 * HIDD].T
    wax = W_a1[:, 2 * HIDD:].T
    wmc = (W_m1[:, :HIDD] - W_m1[:, HIDD:2 * HIDD]).T
    wmn = W_m1[:, HIDD:2 * HIDD].T
    wmx = W_m1[:, 2 * HIDD:].T
    wm2t = W_m2.T
    full = lambda shape: pl.BlockSpec(shape, lambda i: (0, 0))
    return pl.pallas_call(
        _mlp_kernel,
        grid=grid,
        in_specs=[
            pl.BlockSpec((R, HIDD), lambda i: (i, 0)),
            pl.BlockSpec((R * KNB, 3), lambda i: (i, 0)),
            pl.BlockSpec((R * KNB, HIDD), lambda i: (i, 0)),
            full((HIDD, EDGED)),
            full((HIDD, EDGED)),
            full((3, EDGED)),
            full((1, EDGED)),
            full((1, EDGED)),
            full((HIDD, HIDD)),
            full((HIDD, HIDD)),
            full((3, HIDD)),
            full((1, HIDD)),
            full((HIDD, HIDD)),
            full((1, HIDD)),
        ],
        out_specs=pl.BlockSpec((R, HIDD), lambda i: (i, 0)),
        out_shape=jax.ShapeDtypeStruct((N, HIDD), jnp.float32),
        interpret=interpret,
    )(h2, dxyz, neigh, wac, wan, wax, b_a1.reshape(1, EDGED), W_a2,
      wmc, wmn, wmx, b_m1.reshape(1, HIDD), wm2t, b_m2.reshape(1, HIDD))


def kernel(xyz, h, W_a1, b_a1, W_a2, b_a2, W_m1, b_m1, W_m2, b_m2):
    B, N, C = h.shape
    xyz2 = xyz[0]
    h2 = h[0]
    xyzt = xyz2.T

    m, d2 = _chunk_mins(xyz2, xyzt)
    idx, dxyz = _topk_sc(m, d2, xyzt[0], xyzt[1], xyzt[2])
    neigh = _gather_sc(h2, idx.reshape(-1))
    dxyz_e = dxyz.reshape(N, 3, KNB).transpose(0, 2, 1).reshape(N * KNB, 3)

    out = _edge_mlp(h2, dxyz_e, neigh, W_a1, b_a1, W_a2, b_a2,
                    W_m1, b_m1, W_m2, b_m2)
    return out[None]


# speedup vs baseline: 29.4101x; 1.0014x over previous
import dataclasses
import functools
import jax
import jax.numpy as jnp
from jax import lax
from jax.experimental import pallas as pl
from jax.experimental.pallas import tpu as pltpu
from jax.experimental.pallas import tpu_sc as plsc

KNB = 16
HIDD = 128
EDGED = HIDD * 2 + 3
NCHUNK = 512
LANES = 16
NC, NS = 2, 16
NW = NC * NS


def _chunkmin_kernel(xyzb_ref, xyzt_ref, m_ref, d2_ref):
    R = xyzb_ref.shape[0]
    x0 = xyzb_ref[:, 0:1]
    x1 = xyzb_ref[:, 1:2]
    x2 = xyzb_ref[:, 2:3]
    sqr = (x0 * x0 + x1 * x1) + x2 * x2
    y0 = xyzt_ref[0:1, :]
    y1 = xyzt_ref[1:2, :]
    y2 = xyzt_ref[2:3, :]
    sqc = (y0 * y0 + y1 * y1) + y2 * y2
    xb = xyzb_ref[...].astype(jnp.bfloat16)
    yb = xyzt_ref[...].astype(jnp.bfloat16)
    p = jnp.dot(xb, yb, preferred_element_type=jnp.float32)
    d2 = (sqr + sqc) - 2.0 * p
    d2_ref[...] = d2
    m_ref[...] = jnp.min(d2.reshape(R, LANES, NCHUNK), axis=1)


def _chunk_mins(xyz2, xyzt, interpret=False):
    N = xyz2.shape[0]
    R = 256
    return pl.pallas_call(
        _chunkmin_kernel,
        grid=(N // R,),
        in_specs=[
            pl.BlockSpec((R, 3), lambda i: (i, 0)),
            pl.BlockSpec((3, N), lambda i: (0, 0)),
        ],
        out_specs=[
            pl.BlockSpec((R, NCHUNK), lambda i: (i, 0)),
            pl.BlockSpec((R, N), lambda i: (i, 0)),
        ],
        out_shape=[
            jax.ShapeDtypeStruct((N, NCHUNK), jnp.float32),
            jax.ShapeDtypeStruct((N, N), jnp.float32),
        ],
        interpret=interpret,
    )(xyz2, xyzt)


def _merge_sorted(ak, av, bk, bv):
    bk2 = lax.rev(bk, (0,))
    bv2 = lax.rev(bv, (0,))
    take_b = bk2 < ak
    mk = jnp.where(take_b, bk2, ak)
    mv = jnp.where(take_b, bv2, av)
    return plsc.sort_key_val(mk, mv)


def _tree_top16(leaves):
    while len(leaves) > 1:
        nxt = []
        for i in range(0, len(leaves) - 1, 2):
            nxt.append(_merge_sorted(*leaves[i], *leaves[i + 1]))
        if len(leaves) % 2:
            nxt.append(leaves[-1])
        leaves = nxt
    return leaves[0]


def _splat(vec, j):
    dnums = lax.GatherDimensionNumbers(
        offset_dims=(), collapsed_slice_dims=(0,), start_index_map=(0,))
    idxvec = jnp.full((LANES, 1), j, jnp.int32)
    return lax.gather(vec, idxvec, dnums, slice_sizes=(1,),
                      mode=lax.GatherScatterMode.PROMISE_IN_BOUNDS)


def _topk_sc(m, d2, xs, ys, zs):
    N = m.shape[0]
    rows_per = N // NW
    mesh = plsc.VectorSubcoreMesh(core_axis_name="core",
                                  subcore_axis_name="subcore")

    cp = pltpu.CompilerParams()
    if "needs_layout_passes" in pltpu.CompilerParams.__dataclass_fields__:
        cp = dataclasses.replace(cp, needs_layout_passes=False)

    @functools.partial(
        pl.kernel,
        out_type=(jax.ShapeDtypeStruct((N, KNB), jnp.int32),
                  jax.ShapeDtypeStruct((N, 3 * KNB), jnp.float32)),
        mesh=mesh,
        compiler_params=cp,
        scratch_types=[
            pltpu.VMEM((16, NCHUNK), jnp.float32),
            pltpu.VMEM((N,), jnp.float32),
            pltpu.VMEM((N,), jnp.float32),
            pltpu.VMEM((N,), jnp.float32),
            pltpu.VMEM((N,), jnp.float32),
            pltpu.VMEM((N,), jnp.float32),
            pltpu.VMEM((rows_per, KNB), jnp.int32),
            pltpu.VMEM((rows_per, 3 * KNB), jnp.float32),
            pltpu.SemaphoreType.DMA,
            pltpu.SemaphoreType.DMA,
        ],
    )
    def k(m_hbm, d2_hbm, xs_hbm, ys_hbm, zs_hbm, idx_hbm, dxyz_hbm,
          m_v, row_a, row_b, xs_v, ys_v, zs_v, idx_v, dxyz_v, sem_a, sem_b):
        wid = lax.axis_index("subcore") * NC + lax.axis_index("core")
        base = wid * rows_per
        pltpu.sync_copy(xs_hbm, xs_v)
        pltpu.sync_copy(ys_hbm, ys_v)
        pltpu.sync_copy(zs_hbm, zs_v)
        iota = lax.iota(jnp.int32, LANES)

        def process(gl, buf):
            leaves = []
            for c in range(NCHUNK // LANES):
                v = m_v[gl % 16, pl.ds(c * LANES, LANES)]
                leaves.append(plsc.sort_key_val(v, c * LANES + iota))
            rk, rv = _tree_top16(leaves)
            leaves = []
            for j in range(KNB):
                cj = _splat(rv, j)
                cols = cj + NCHUNK * iota
                d2v = plsc.load_gather(buf, [cols])
                leaves.append(plsc.sort_key_val(d2v, cols))
            fk, fv = _tree_top16(leaves)
            idx_v[gl, :] = fv
            r = base + gl
            rfull = jnp.full((LANES,), r, jnp.int32)
            cx = plsc.load_gather(xs_v, [rfull])
            cy = plsc.load_gather(ys_v, [rfull])
            cz = plsc.load_gather(zs_v, [rfull])
            gx = plsc.load_gather(xs_v, [fv])
            gy = plsc.load_gather(ys_v, [fv])
            gz = plsc.load_gather(zs_v, [fv])
            dxyz_v[gl, pl.ds(0, LANES)] = gx - cx
            dxyz_v[gl, pl.ds(LANES, LANES)] = gy - cy
            dxyz_v[gl, pl.ds(2 * LANES, LANES)] = gz - cz

        @pl.loop(0, rows_per // 16)
        def _(t):
            pltpu.sync_copy(m_hbm.at[pl.ds(base + t * 16, 16)], m_v)
            pltpu.async_copy(d2_hbm.at[base + t * 16], row_a, sem_a)

            @pl.loop(0, 8)
            def _(p):
                g0 = t * 16 + 2 * p
                r0 = base + g0
                pltpu.make_async_copy(d2_hbm.at[r0], row_a, sem_a).wait()
                pltpu.async_copy(d2_hbm.at[r0 + 1], row_b, sem_b)
                process(g0, row_a)
                pltpu.make_async_copy(d2_hbm.at[r0 + 1], row_b, sem_b).wait()

                @pl.when(2 * p + 2 < 16)
                def _():
                    pltpu.async_copy(d2_hbm.at[r0 + 2], row_a, sem_a)

                process(g0 + 1, row_b)

        pltpu.sync_copy(idx_v, idx_hbm.at[pl.ds(base, rows_per)])
        pltpu.sync_copy(dxyz_v, dxyz_hbm.at[pl.ds(base, rows_per)])

    return k(m, d2, xs, ys, zs)


def _gather_sc(table, idx_flat):
    NE = idx_flat.shape[0]
    D = table.shape[1]
    W = 128
    mesh = plsc.VectorSubcoreMesh(core_axis_name="core",
                                  subcore_axis_name="subcore")
    idx2 = idx_flat.reshape(1, NE)

    @functools.partial(
        pl.kernel,
        out_type=jax.ShapeDtypeStruct((NE, D), table.dtype),
        mesh=mesh,
    )
    def k(tab_hbm, i_hbm, o_hbm):
        def body(i_vmem, o_vmem):
            pltpu.sync_copy(tab_hbm.at[i_vmem.at[0]], o_vmem)

        pltpu.emit_pipeline(
            body,
            grid=(NE // W,),
            in_specs=[pl.BlockSpec((1, W), lambda i: (0, i))],
            out_specs=[pl.BlockSpec((W, D), lambda i: (i, 0))],
            core_axis_name=("core", "subcore"),
            dimension_semantics=(pltpu.PARALLEL,),
        )(i_hbm, o_hbm)

    return k(table, idx2)


def _mlp_kernel(h_ref, dxyz_ref, neigh_ref, wac_ref, wan_ref, wax_ref,
                ba1_ref, wa2_ref, wmc_ref, wmn_ref, wmx_ref, bm1_ref,
                wm2t_ref, bm2_ref, out_ref):
    R = h_ref.shape[0]
    E = R * KNB
    f32 = jnp.float32
    c_h = h_ref[...]
    n_h = neigh_ref[...]
    dr = dxyz_ref[...]
    dxyz = [dr[:, d:d + 1] for d in range(3)]

    pa = jnp.dot(c_h, wac_ref[...], preferred_element_type=f32)
    a_pre = (jnp.broadcast_to(pa[:, None, :], (R, KNB, EDGED)).reshape(E, EDGED)
             + jnp.dot(n_h, wan_ref[...], preferred_element_type=f32)
             + ba1_ref[...])
    for d in range(3):
        a_pre = a_pre + dxyz[d] * wax_ref[d:d + 1, :]
    a = jnp.maximum(a_pre, 0.0)
    scores = jnp.sum(a * wa2_ref[...], axis=1).reshape(R, KNB)
    smax = jnp.max(scores, axis=1, keepdims=True)
    sexp = jnp.exp(scores - smax)
    alpha = sexp / jnp.sum(sexp, axis=1, keepdims=True)

    pm = jnp.dot(c_h, wmc_ref[...], preferred_element_type=f32)
    m_pre = (jnp.broadcast_to(pm[:, None, :], (R, KNB, HIDD)).reshape(E, HIDD)
             + jnp.dot(n_h, wmn_ref[...], preferred_element_type=f32)
             + bm1_ref[...])
    for d in range(3):
        m_pre = m_pre + dxyz[d] * wmx_ref[d:d + 1, :]
    m = jnp.maximum(m_pre, 0.0)
    msg_flat = jnp.dot(m, wm2t_ref[...], preferred_element_type=f32) + bm2_ref[...]
    wmsg = msg_flat * alpha.reshape(E, 1)
    msg = jnp.sum(wmsg.reshape(R, KNB, HIDD), axis=1)
    out_ref[...] = c_h + msg


def _edge_mlp(h2, dxyz, neigh, W_a1, b_a1, W_a2, b_a2, W_m1, b_m1, W_m2,
              b_m2, interpret=False):
    N = h2.shape[0]
    R = 256
    grid = (N // R,)
    wac = (W_a1[:, :HIDD] - W_a1[:, HIDD:2 * HIDD]).T
    wan = W_a1[:, HIDD:2 * HIDD].T
    wax = W_a1[:, 2 * HIDD:].T
    wmc = (W_m1[:, :HIDD] - W_m1[:, HIDD:2 * HIDD]).T
    wmn = W_m1[:, HIDD:2 * HIDD].T
    wmx = W_m1[:, 2 * HIDD:].T
    wm2t = W_m2.T
    full = lambda shape: pl.BlockSpec(shape, lambda i: (0, 0))
    return pl.pallas_call(
        _mlp_kernel,
        grid=grid,
        in_specs=[
            pl.BlockSpec((R, HIDD), lambda i: (i, 0)),
            pl.BlockSpec((R * KNB, 3), lambda i: (i, 0)),
            pl.BlockSpec((R * KNB, HIDD), lambda i: (i, 0)),
            full((HIDD, EDGED)),
            full((HIDD, EDGED)),
            full((3, EDGED)),
            full((1, EDGED)),
            full((1, EDGED)),
            full((HIDD, HIDD)),
            full((HIDD, HIDD)),
            full((3, HIDD)),
            full((1, HIDD)),
            full((HIDD, HIDD)),
            full((1, HIDD)),
        ],
        out_specs=pl.BlockSpec((R, HIDD), lambda i: (i, 0)),
        out_shape=jax.ShapeDtypeStruct((N, HIDD), jnp.float32),
        interpret=interpret,
    )(h2, dxyz, neigh, wac, wan, wax, b_a1.reshape(1, EDGED), W_a2,
      wmc, wmn, wmx, b_m1.reshape(1, HIDD), wm2t, b_m2.reshape(1, HIDD))


def kernel(xyz, h, W_a1, b_a1, W_a2, b_a2, W_m1, b_m1, W_m2, b_m2):
    B, N, C = h.shape
    xyz2 = xyz[0]
    h2 = h[0]
    xyzt = xyz2.T

    m, d2 = _chunk_mins(xyz2, xyzt)
    idx, dxyz = _topk_sc(m, d2, xyzt[0], xyzt[1], xyzt[2])
    neigh = _gather_sc(h2, idx.reshape(-1))
    dxyz_e = dxyz.reshape(N, 3, KNB).transpose(0, 2, 1).reshape(N * KNB, 3)

    out = _edge_mlp(h2, dxyz_e, neigh, W_a1, b_a1, W_a2, b_a2,
                    W_m1, b_m1, W_m2, b_m2)
    return out[None]
